# baseline (speedup 1.0000x reference)
import functools

import jax
import jax.numpy as jnp
from jax import lax
from jax.experimental import pallas as pl
from jax.experimental.pallas import tpu as pltpu
from jax.experimental.pallas import tpu_sc as plsc

PAD_ID = 0
D = 64
DP = 128
NC, NS = 2, 16
NW = NC * NS
NBUF = 2
NIDX = 4


@functools.partial(jax.jit, static_argnames=("b", "l"))
def _gather_both(lab_idx, pos_idx, lab_tab_pad, pos_tab_pad, b, l):
    bpw = b // NW
    c0 = 96
    c1 = l - c0
    cmax = max(c0, c1)
    lpad = 2 * 128
    mesh = plsc.VectorSubcoreMesh(core_axis_name="c", subcore_axis_name="s")

    @functools.partial(
        pl.kernel,
        mesh=mesh,
        out_type=(
            jax.ShapeDtypeStruct((b * l, D), jnp.float32),
            jax.ShapeDtypeStruct((b * l, D), jnp.float32),
        ),
        scratch_types=[
            pltpu.VMEM((NIDX * lpad,), jnp.int32),
            pltpu.VMEM((NIDX * lpad,), jnp.int32),
            pltpu.VMEM((NBUF, cmax, DP), jnp.float32),
            pltpu.VMEM((NBUF, cmax, DP), jnp.float32),
            pltpu.VMEM((NBUF, cmax, D), jnp.float32),
            pltpu.VMEM((NBUF, cmax, D), jnp.float32),
            pltpu.VMEM_SHARED((1000, D), jnp.float32),
            pltpu.SemaphoreType.DMA,
            pltpu.SemaphoreType.DMA,
            pltpu.SemaphoreType.DMA,
            pltpu.SemaphoreType.DMA,
            pltpu.SemaphoreType.DMA,
            pltpu.SemaphoreType.DMA,
        ],
    )
    def body(lab_idx_hbm, pos_idx_hbm, lab_tab_hbm, pos_tab_hbm,
             pos_out_hbm, lab_out_hbm,
             lab_idx_v, pos_idx_v, lab_rows, pos_rows, lab_pk, pos_pk,
             pos_shared,
             sem_il, sem_ip, sem_gl, sem_gp, sem_wl, sem_wp):
        sid = lax.axis_index("s")
        wid = lax.axis_index("c") * NS + sid
        base = wid * bpw
        nt = 2 * bpw

        @pl.when(sid == 0)
        def _():
            npos = pos_tab_hbm.shape[0]
            done = 0
            while done < npos:
                sz = min(cmax, npos - done)
                pltpu.sync_copy(pos_tab_hbm.at[pl.ds(done, sz)],
                                pos_rows.at[0, pl.ds(0, sz)])

                def crow(j, carry, sz=sz):
                    for k in range(4):
                        pos_pk[0, j, pl.ds(16 * k, 16)] = (
                            pos_rows[0, j, pl.ds(16 * k, 16)])
                    return carry

                lax.fori_loop(0, sz, crow, 0)
                pltpu.sync_copy(pos_pk.at[0, pl.ds(0, sz)],
                                pos_shared.at[pl.ds(done, sz)])
                done += sz

        plsc.subcore_barrier()

        def clen(t):
            return lax.select(lax.rem(t, 2) == 0, c0, c1)

        def idx_copy(i):
            q0 = pl.multiple_of(lax.rem(i, NIDX) * lpad, lpad)
            off = (base + i) * l
            return (
                pltpu.make_async_copy(
                    lab_idx_hbm.at[pl.ds(off, l)],
                    lab_idx_v.at[pl.ds(q0, l)], sem_il),
                pltpu.make_async_copy(
                    pos_idx_hbm.at[pl.ds(off, l)],
                    pos_idx_v.at[pl.ds(q0, l)], sem_ip),
            )

        def gather(t, c):
            i = lax.div(t, 2)
            h = lax.rem(t, 2)
            q0 = pl.multiple_of(lax.rem(i, NIDX) * lpad, lpad) + h * c0
            r = lax.rem(t, NBUF)
            return (
                pltpu.make_async_copy(
                    lab_tab_hbm.at[lab_idx_v.at[pl.ds(q0, c)]],
                    lab_rows.at[r, pl.ds(0, c)], sem_gl),
                pltpu.make_async_copy(
                    pos_shared.at[pos_idx_v.at[pl.ds(q0, c)]],
                    pos_pk.at[r, pl.ds(0, c)], sem_gp),
            )

        def repack(t, c):
            r = lax.rem(t, NBUF)

            def row(j, carry):
                for k in range(4):
                    lab_pk[r, j, pl.ds(16 * k, 16)] = (
                        lab_rows[r, j, pl.ds(16 * k, 16)])
                return carry

            lax.fori_loop(0, c, row, 0)

        def writeback(t, c):
            i = lax.div(t, 2)
            h = lax.rem(t, 2)
            r = lax.rem(t, NBUF)
            off = (base + i) * l + h * c0
            return (
                pltpu.make_async_copy(
                    lab_pk.at[r, pl.ds(0, c)],
                    lab_out_hbm.at[pl.ds(off, c)], sem_wl),
                pltpu.make_async_copy(
                    pos_pk.at[r, pl.ds(0, c)],
                    pos_out_hbm.at[pl.ds(off, c)], sem_wp),
            )

        for i in range(NBUF):
            for cc in idx_copy(i):
                cc.start()

        def run_gather(t):
            @pl.when(lax.rem(t, 2) == 0)
            def _():
                for cc in gather(t, c0):
                    cc.start()

            @pl.when(lax.rem(t, 2) == 1)
            def _():
                for cc in gather(t, c1):
                    cc.start()

        def wait_gather(t):
            @pl.when(lax.rem(t, 2) == 0)
            def _():
                for cc in gather(t, c0):
                    cc.wait()

            @pl.when(lax.rem(t, 2) == 1)
            def _():
                for cc in gather(t, c1):
                    cc.wait()

        def run_repack(t):
            @pl.when(lax.rem(t, 2) == 0)
            def _():
                repack(t, c0)

            @pl.when(lax.rem(t, 2) == 1)
            def _():
                repack(t, c1)

        def run_writeback(t):
            @pl.when(lax.rem(t, 2) == 0)
            def _():
                for cc in writeback(t, c0):
                    cc.start()

            @pl.when(lax.rem(t, 2) == 1)
            def _():
                for cc in writeback(t, c1):
                    cc.start()

        def wait_writeback(t):
            @pl.when(lax.rem(t, 2) == 0)
            def _():
                for cc in writeback(t, c0):
                    cc.wait()

            @pl.when(lax.rem(t, 2) == 1)
            def _():
                for cc in writeback(t, c1):
                    cc.wait()

        def step(t, carry):
            i = lax.div(t, 2)

            @pl.when(lax.rem(t, 2) == 0)
            def _():
                for cc in idx_copy(i):
                    cc.wait()

            @pl.when(t >= NBUF)
            def _():
                wait_writeback(t - NBUF)

            run_gather(t)

            @pl.when((lax.rem(t, 2) == 0) & (i + NBUF < bpw))
            def _():
                for cc in idx_copy(i + NBUF):
                    cc.start()

            @pl.when(t >= 1)
            def _():
                wait_gather(t - 1)
                run_repack(t - 1)
                run_writeback(t - 1)

            return carry

        lax.fori_loop(0, nt, step, 0)
        wait_gather(nt - 1)
        run_repack(nt - 1)
        run_writeback(nt - 1)
        for t in range(nt - NBUF, nt):
            wait_writeback(t)

    return body(lab_idx, pos_idx, lab_tab_pad, pos_tab_pad)


def kernel(label_ids, pos_ids, label_table, pos_table):
    b, l = label_ids.shape
    pos_table = pos_table.at[PAD_ID].set(0.0)
    lab_tab_pad = jnp.pad(label_table, ((0, 0), (0, DP - D)))
    pos_tab_pad = jnp.pad(pos_table, ((0, 0), (0, DP - D)))
    lab_idx = label_ids.reshape(-1).astype(jnp.int32)
    pos_idx = pos_ids.reshape(-1).astype(jnp.int32)
    pos_out, lab_out = _gather_both(
        lab_idx, pos_idx, lab_tab_pad, pos_tab_pad, b, l)
    return pos_out.reshape(b, l, D), lab_out.reshape(b, l, D)


# speedup vs baseline: 1.0185x; 1.0185x over previous
import functools

import jax
import jax.numpy as jnp
from jax import lax
from jax.experimental import pallas as pl
from jax.experimental.pallas import tpu as pltpu
from jax.experimental.pallas import tpu_sc as plsc

PAD_ID = 0
D = 64
DP = 128
NC, NS = 2, 16
NW = NC * NS
NBUF = 2
NIDX = 4


@functools.partial(jax.jit, static_argnames=("b", "l"))
def _gather_both(lab_idx, pos_idx, lab_tab_pad, pos_tab_pad, b, l):
    bpw = b // NW
    c0 = 96
    c1 = l - c0
    cmax = max(c0, c1)
    lpad = 2 * 128
    mesh = plsc.VectorSubcoreMesh(core_axis_name="c", subcore_axis_name="s")

    @functools.partial(
        pl.kernel,
        mesh=mesh,
        out_type=(
            jax.ShapeDtypeStruct((b * l, D), jnp.float32),
            jax.ShapeDtypeStruct((b * l, D), jnp.float32),
        ),
        scratch_types=[
            pltpu.VMEM((NIDX * lpad,), jnp.int32),
            pltpu.VMEM((NIDX * lpad,), jnp.int32),
            pltpu.VMEM((NBUF, cmax, DP), jnp.float32),
            pltpu.VMEM((NBUF, cmax, DP), jnp.float32),
            pltpu.VMEM((NBUF, cmax, D), jnp.float32),
            pltpu.VMEM((NBUF, cmax, D), jnp.float32),
            pltpu.VMEM_SHARED((1000, DP), jnp.float32),
            pltpu.SemaphoreType.DMA,
            pltpu.SemaphoreType.DMA,
            pltpu.SemaphoreType.DMA,
            pltpu.SemaphoreType.DMA,
            pltpu.SemaphoreType.DMA,
            pltpu.SemaphoreType.DMA,
        ],
    )
    def body(lab_idx_hbm, pos_idx_hbm, lab_tab_hbm, pos_tab_hbm,
             pos_out_hbm, lab_out_hbm,
             lab_idx_v, pos_idx_v, lab_rows, pos_rows, lab_pk, pos_pk,
             pos_shared,
             sem_il, sem_ip, sem_gl, sem_gp, sem_wl, sem_wp):
        sid = lax.axis_index("s")
        wid = lax.axis_index("c") * NS + sid
        base = wid * bpw
        nt = 2 * bpw

        @pl.when(sid == 0)
        def _():
            pltpu.sync_copy(pos_tab_hbm, pos_shared)

        plsc.subcore_barrier()

        def clen(t):
            return lax.select(lax.rem(t, 2) == 0, c0, c1)

        def idx_copy(i):
            q0 = pl.multiple_of(lax.rem(i, NIDX) * lpad, lpad)
            off = (base + i) * l
            return (
                pltpu.make_async_copy(
                    lab_idx_hbm.at[pl.ds(off, l)],
                    lab_idx_v.at[pl.ds(q0, l)], sem_il),
                pltpu.make_async_copy(
                    pos_idx_hbm.at[pl.ds(off, l)],
                    pos_idx_v.at[pl.ds(q0, l)], sem_ip),
            )

        def gather(t, c):
            i = lax.div(t, 2)
            h = lax.rem(t, 2)
            q0 = pl.multiple_of(lax.rem(i, NIDX) * lpad, lpad) + h * c0
            r = lax.rem(t, NBUF)
            return (
                pltpu.make_async_copy(
                    lab_tab_hbm.at[lab_idx_v.at[pl.ds(q0, c)]],
                    lab_rows.at[r, pl.ds(0, c)], sem_gl),
                pltpu.make_async_copy(
                    pos_shared.at[pos_idx_v.at[pl.ds(q0, c)]],
                    pos_rows.at[r, pl.ds(0, c)], sem_gp),
            )

        def repack(t, c):
            r = lax.rem(t, NBUF)

            def row(j, carry):
                for k in range(4):
                    lab_pk[r, j, pl.ds(16 * k, 16)] = (
                        lab_rows[r, j, pl.ds(16 * k, 16)])
                    pos_pk[r, j, pl.ds(16 * k, 16)] = (
                        pos_rows[r, j, pl.ds(16 * k, 16)])
                return carry

            lax.fori_loop(0, c, row, 0)

        def writeback(t, c):
            i = lax.div(t, 2)
            h = lax.rem(t, 2)
            r = lax.rem(t, NBUF)
            off = (base + i) * l + h * c0
            return (
                pltpu.make_async_copy(
                    lab_pk.at[r, pl.ds(0, c)],
                    lab_out_hbm.at[pl.ds(off, c)], sem_wl),
                pltpu.make_async_copy(
                    pos_pk.at[r, pl.ds(0, c)],
                    pos_out_hbm.at[pl.ds(off, c)], sem_wp),
            )

        for i in range(NBUF):
            for cc in idx_copy(i):
                cc.start()

        def run_gather(t):
            @pl.when(lax.rem(t, 2) == 0)
            def _():
                for cc in gather(t, c0):
                    cc.start()

            @pl.when(lax.rem(t, 2) == 1)
            def _():
                for cc in gather(t, c1):
                    cc.start()

        def wait_gather(t):
            @pl.when(lax.rem(t, 2) == 0)
            def _():
                for cc in gather(t, c0):
                    cc.wait()

            @pl.when(lax.rem(t, 2) == 1)
            def _():
                for cc in gather(t, c1):
                    cc.wait()

        def run_repack(t):
            @pl.when(lax.rem(t, 2) == 0)
            def _():
                repack(t, c0)

            @pl.when(lax.rem(t, 2) == 1)
            def _():
                repack(t, c1)

        def run_writeback(t):
            @pl.when(lax.rem(t, 2) == 0)
            def _():
                for cc in writeback(t, c0):
                    cc.start()

            @pl.when(lax.rem(t, 2) == 1)
            def _():
                for cc in writeback(t, c1):
                    cc.start()

        def wait_writeback(t):
            @pl.when(lax.rem(t, 2) == 0)
            def _():
                for cc in writeback(t, c0):
                    cc.wait()

            @pl.when(lax.rem(t, 2) == 1)
            def _():
                for cc in writeback(t, c1):
                    cc.wait()

        def step(t, carry):
            i = lax.div(t, 2)

            @pl.when(lax.rem(t, 2) == 0)
            def _():
                for cc in idx_copy(i):
                    cc.wait()

            @pl.when(t >= NBUF)
            def _():
                wait_writeback(t - NBUF)

            run_gather(t)

            @pl.when((lax.rem(t, 2) == 0) & (i + NBUF < bpw))
            def _():
                for cc in idx_copy(i + NBUF):
                    cc.start()

            @pl.when(t >= 1)
            def _():
                wait_gather(t - 1)
                run_repack(t - 1)
                run_writeback(t - 1)

            return carry

        lax.fori_loop(0, nt, step, 0)
        wait_gather(nt - 1)
        run_repack(nt - 1)
        run_writeback(nt - 1)
        for t in range(nt - NBUF, nt):
            wait_writeback(t)

    return body(lab_idx, pos_idx, lab_tab_pad, pos_tab_pad)


def kernel(label_ids, pos_ids, label_table, pos_table):
    b, l = label_ids.shape
    pos_table = pos_table.at[PAD_ID].set(0.0)
    lab_tab_pad = jnp.pad(label_table, ((0, 0), (0, DP - D)))
    pos_tab_pad = jnp.pad(pos_table, ((0, 0), (0, DP - D)))
    lab_idx = label_ids.reshape(-1).astype(jnp.int32)
    pos_idx = pos_ids.reshape(-1).astype(jnp.int32)
    pos_out, lab_out = _gather_both(
        lab_idx, pos_idx, lab_tab_pad, pos_tab_pad, b, l)
    return pos_out.reshape(b, l, D), lab_out.reshape(b, l, D)
